# baseline (speedup 1.0000x reference)
import functools

import jax
import jax.numpy as jnp
from jax import lax
from jax.experimental import pallas as pl
from jax.experimental.pallas import tpu as pltpu
from jax.experimental.pallas import tpu_sc as plsc

B, C, H, W = 4, 96, 512, 512
TH, TW = 448, 448
NC, NS = 2, 16
NW = NC * NS
CH_PER_W = (B * C) // NW
CR = 64
NCHUNK = TH // CR
RW = TW + 8


def _crop_body(rows_hbm, meta_hbm, out_hbm, meta_v, in0, in1, ob0, ob1, rsems, wsems):
    ibufs = (in0, in1)
    obufs = (ob0, ob1)
    wid = lax.axis_index("s") * NC + lax.axis_index("c")
    pltpu.sync_copy(meta_hbm, meta_v)

    def channel(t, carry):
        bc = wid * CH_PER_W + t
        b = bc // C
        bvec = jnp.zeros((16,), jnp.int32) + b
        i_s = jnp.max(plsc.load_gather(meta_v, [bvec]))
        j_s = jnp.max(plsc.load_gather(meta_v, [bvec + 4]))
        j_al = pl.multiple_of(j_s & ~7, 8)
        j_off = j_s & 7
        row0 = bc * H + i_s
        orow0 = bc * TH

        def read(k):
            return pltpu.async_copy(
                rows_hbm.at[pl.ds(row0 + k * CR, CR), pl.ds(j_al, RW)],
                ibufs[k % 2],
                rsems.at[k % 2],
            )

        def write(k):
            return pltpu.async_copy(
                obufs[k % 2],
                out_hbm.at[pl.ds(orow0 + k * CR, CR)],
                wsems.at[k % 2],
            )

        rh = {0: read(0), 1: read(1)}
        wh = {}
        for k in range(NCHUNK):
            rh[k].wait()
            if k >= 2:
                wh[k - 2].wait()
            else:
                @pl.when(t > 0)
                def _drain():
                    pltpu.make_async_copy(
                        obufs[k % 2],
                        out_hbm.at[pl.ds(orow0, CR)],
                        wsems.at[k % 2],
                    ).wait()

            ibuf = ibufs[k % 2]
            obuf = obufs[k % 2]

            @plsc.parallel_loop(0, CR, step=1, unroll=2)
            def shift_row(r):
                for tt in range(TW // 16):
                    v = ibuf[r, pl.ds(j_off + 16 * tt, 16)]
                    obuf[r, pl.ds(16 * tt, 16)] = v

            if k + 2 < NCHUNK:
                rh[k + 2] = read(k + 2)
            wh[k] = write(k)
        return carry

    lax.fori_loop(0, CH_PER_W, channel, 0)

    for s in range(2):
        pltpu.make_async_copy(
            obufs[s], out_hbm.at[pl.ds(0, CR)], wsems.at[s]
        ).wait()


def kernel(img, i, j):
    rows = img.reshape(B * C * H, W)
    meta = jnp.concatenate(
        [i.astype(jnp.int32), j.astype(jnp.int32), jnp.zeros((8,), jnp.int32)]
    )
    mesh = plsc.VectorSubcoreMesh(core_axis_name="c", subcore_axis_name="s")
    out = pl.kernel(
        _crop_body,
        mesh=mesh,
        out_type=jax.ShapeDtypeStruct((B * C * TH, TW), jnp.float32),
        scratch_types=[
            pltpu.VMEM((16,), jnp.int32),
            pltpu.VMEM((CR, RW), jnp.float32),
            pltpu.VMEM((CR, RW), jnp.float32),
            pltpu.VMEM((CR, TW), jnp.float32),
            pltpu.VMEM((CR, TW), jnp.float32),
            pltpu.SemaphoreType.DMA((2,)),
            pltpu.SemaphoreType.DMA((2,)),
        ],
        compiler_params=pltpu.CompilerParams(
            use_tc_tiling_on_sc=False, needs_layout_passes=False
        ),
    )(rows, meta)
    return out.reshape(B, C, TH, TW)


# speedup vs baseline: 1.0050x; 1.0050x over previous
import functools

import jax
import jax.numpy as jnp
from jax import lax
from jax.experimental import pallas as pl
from jax.experimental.pallas import tpu as pltpu
from jax.experimental.pallas import tpu_sc as plsc

B, C, H, W = 4, 96, 512, 512
TH, TW = 448, 448
NC, NS = 2, 16
NW = NC * NS
CH_PER_W = (B * C) // NW
CR = 64
NCHUNK = TH // CR
RW = TW + 8


def _crop_body(rows_hbm, meta_hbm, out_hbm, meta_v, in0, in1, ob0, ob1, rsems, wsems):
    ibufs = (in0, in1)
    obufs = (ob0, ob1)
    wid = lax.axis_index("s") * NC + lax.axis_index("c")
    pltpu.sync_copy(meta_hbm, meta_v)

    def channel(t, carry):
        bc = wid * CH_PER_W + t
        b = bc // C
        bvec = jnp.zeros((16,), jnp.int32) + b
        i_s = jnp.max(plsc.load_gather(meta_v, [bvec]))
        j_s = jnp.max(plsc.load_gather(meta_v, [bvec + 4]))
        j_al = pl.multiple_of(jnp.minimum(j_s & ~7, W - RW), 8)
        j_off = j_s - j_al
        row0 = bc * H + i_s
        orow0 = bc * TH

        def read(k):
            return pltpu.async_copy(
                rows_hbm.at[pl.ds(row0 + k * CR, CR), pl.ds(j_al, RW)],
                ibufs[k % 2],
                rsems.at[k % 2],
            )

        def write(k):
            return pltpu.async_copy(
                obufs[k % 2],
                out_hbm.at[pl.ds(orow0 + k * CR, CR)],
                wsems.at[k % 2],
            )

        rh = {0: read(0), 1: read(1)}
        wh = {}
        for k in range(NCHUNK):
            rh[k].wait()
            if k >= 2:
                wh[k - 2].wait()
            else:
                @pl.when(t > 0)
                def _drain():
                    pltpu.make_async_copy(
                        obufs[k % 2],
                        out_hbm.at[pl.ds(orow0, CR)],
                        wsems.at[k % 2],
                    ).wait()

            ibuf = ibufs[k % 2]
            obuf = obufs[k % 2]

            @plsc.parallel_loop(0, CR, step=1, unroll=2)
            def shift_row(r):
                for tt in range(TW // 16):
                    v = ibuf[r, pl.ds(j_off + 16 * tt, 16)]
                    obuf[r, pl.ds(16 * tt, 16)] = v

            if k + 2 < NCHUNK:
                rh[k + 2] = read(k + 2)
            wh[k] = write(k)
        return carry

    lax.fori_loop(0, CH_PER_W, channel, 0)

    for s in range(2):
        pltpu.make_async_copy(
            obufs[s], out_hbm.at[pl.ds(0, CR)], wsems.at[s]
        ).wait()


def kernel(img, i, j):
    rows = img.reshape(B * C * H, W)
    meta = jnp.concatenate(
        [i.astype(jnp.int32), j.astype(jnp.int32), jnp.zeros((8,), jnp.int32)]
    )
    mesh = plsc.VectorSubcoreMesh(core_axis_name="c", subcore_axis_name="s")
    out = pl.kernel(
        _crop_body,
        mesh=mesh,
        out_type=jax.ShapeDtypeStruct((B * C * TH, TW), jnp.float32),
        scratch_types=[
            pltpu.VMEM((16,), jnp.int32),
            pltpu.VMEM((CR, RW), jnp.float32),
            pltpu.VMEM((CR, RW), jnp.float32),
            pltpu.VMEM((CR, TW), jnp.float32),
            pltpu.VMEM((CR, TW), jnp.float32),
            pltpu.SemaphoreType.DMA((2,)),
            pltpu.SemaphoreType.DMA((2,)),
        ],
        compiler_params=pltpu.CompilerParams(
            use_tc_tiling_on_sc=False, needs_layout_passes=False
        ),
    )(rows, meta)
    return out.reshape(B, C, TH, TW)
